# baseline (speedup 1.0000x reference)
import functools

import jax
import jax.numpy as jnp
from jax import lax
from jax.experimental import pallas as pl
from jax.experimental.pallas import tpu as pltpu
from jax.experimental.pallas import tpu_sc as plsc

ALPHA = 0.2
BATCH = 16384
EMBED_DIM = 64
CHUNK = 128


def _sc_body(e_hbm, rc_hbm, ro_hbm, s_hbm, r_hbm, o_hbm, out_hbm,
             s_i, r_i, o_i, srow, orow, rcrow, rorow, outbuf, sem):
    info = plsc.get_sparse_core_info()
    nw = info.num_cores * info.num_subcores
    b_per_w = BATCH // nw
    nchunk = b_per_w // CHUNK

    wid = lax.axis_index("s") * info.num_cores + lax.axis_index("c")
    base = wid * b_per_w

    def chunk_body(c, carry):
        off0 = base + c * CHUNK
        pltpu.sync_copy(s_hbm.at[pl.ds(off0, CHUNK)], s_i)
        pltpu.sync_copy(r_hbm.at[pl.ds(off0, CHUNK)], r_i)
        pltpu.sync_copy(o_hbm.at[pl.ds(off0, CHUNK)], o_i)
        cp_s = pltpu.async_copy(e_hbm.at[s_i], srow, sem)
        cp_o = pltpu.async_copy(e_hbm.at[o_i], orow, sem)
        cp_rc = pltpu.async_copy(rc_hbm.at[r_i], rcrow, sem)
        cp_ro = pltpu.async_copy(ro_hbm.at[r_i], rorow, sem)
        cp_s.wait()
        cp_o.wait()
        cp_rc.wait()
        cp_ro.wait()

        lanes = lax.iota(jnp.int32, 16)

        def group_body(g, carry2):
            outv = jnp.zeros((16,), jnp.float32)
            for j in range(16):
                i = g * 16 + j
                acc = jnp.zeros((16,), jnp.float32)
                for k in range(EMBED_DIM // 16):
                    sl = pl.ds(k * 16, 16)
                    t = orow[i, sl] - srow[i, sl] - rcrow[i, sl]
                    off = jnp.maximum(rorow[i, sl], 0.0)
                    a = jnp.abs(t)
                    dout = jnp.maximum(a - off, 0.0)
                    din = jnp.minimum(a, off)
                    acc = acc + (dout + ALPHA * din)
                tot = jnp.sum(acc)
                outv = jnp.where(lanes == j, -tot, outv)
            outbuf[pl.ds(c * CHUNK + g * 16, 16)] = outv
            return carry2

        lax.fori_loop(0, CHUNK // 16, group_body, 0)
        return carry

    lax.fori_loop(0, nchunk, chunk_body, 0)
    pltpu.sync_copy(outbuf, out_hbm.at[pl.ds(base, b_per_w)])


def kernel(E_center, R_center, R_offset, s, r, o):
    info = plsc.get_sparse_core_info()
    nw = info.num_cores * info.num_subcores
    b_per_w = BATCH // nw

    run = functools.partial(
        pl.kernel,
        out_type=jax.ShapeDtypeStruct((BATCH,), jnp.float32),
        mesh=plsc.VectorSubcoreMesh(core_axis_name="c", subcore_axis_name="s"),
        compiler_params=pltpu.CompilerParams(
            needs_layout_passes=False, use_tc_tiling_on_sc=False),
        scratch_types=[
            pltpu.VMEM((CHUNK,), jnp.int32),
            pltpu.VMEM((CHUNK,), jnp.int32),
            pltpu.VMEM((CHUNK,), jnp.int32),
            pltpu.VMEM((CHUNK, EMBED_DIM), jnp.float32),
            pltpu.VMEM((CHUNK, EMBED_DIM), jnp.float32),
            pltpu.VMEM((CHUNK, EMBED_DIM), jnp.float32),
            pltpu.VMEM((CHUNK, EMBED_DIM), jnp.float32),
            pltpu.VMEM((b_per_w,), jnp.float32),
            pltpu.SemaphoreType.DMA,
        ],
    )(_sc_body)

    return run(E_center, R_center, R_offset,
               s.astype(jnp.int32), r.astype(jnp.int32), o.astype(jnp.int32))
